# Initial kernel scaffold; baseline (speedup 1.0000x reference)
import jax
import jax.numpy as jnp
from jax.experimental import pallas as pl


def kernel(x, emb, W, b):
    raise NotImplementedError("write your pallas kernel here")


# speedup vs baseline: 51.6236x; 51.6236x over previous
import functools

import jax
import jax.numpy as jnp
from jax import lax
from jax.experimental import pallas as pl
from jax.experimental.pallas import tpu as pltpu
from jax.experimental.pallas import tpu_sc as plsc

_VOCAB = 1000
_DIM = 128
_BATCH = 4096
_SEQ = 200
_NCLASS = 3
_CPAD = 8

_NC, _NS = 2, 16
_NW = _NC * _NS
_ROWS = _BATCH // _NW
_GROUPS = _ROWS // 16


def _table_body(emb_ref, w_ref, b_ref, out_ref):
    acc = jnp.dot(emb_ref[...], w_ref[...],
                  preferred_element_type=jnp.float32,
                  precision=lax.Precision.HIGHEST)
    out_ref[...] = (acc + b_ref[...]) * (1.0 / _SEQ)


def _make_table(emb, W, b):
    w_pad = jnp.zeros((_DIM, _CPAD), jnp.float32).at[:, :_NCLASS].set(W.T)
    b_pad = jnp.zeros((1, _CPAD), jnp.float32).at[0, :_NCLASS].set(b)
    return pl.pallas_call(
        _table_body,
        out_shape=jax.ShapeDtypeStruct((_VOCAB, _CPAD), jnp.float32),
    )(emb, w_pad, b_pad)


def _sc_body(x_hbm, t0_hbm, t1_hbm, t2_hbm,
             o0_hbm, o1_hbm, o2_hbm,
             xv, t0v, t1v, t2v, o0v, o1v, o2v):
    wid = lax.axis_index("s") * _NC + lax.axis_index("c")
    base = wid * _ROWS

    pltpu.sync_copy(x_hbm.at[pl.ds(base * _SEQ, _ROWS * _SEQ)], xv)
    pltpu.sync_copy(t0_hbm, t0v)
    pltpu.sync_copy(t1_hbm, t1v)
    pltpu.sync_copy(t2_hbm, t2v)

    for g in range(_GROUPS):
        pos_vec = (lax.iota(jnp.int32, 16) + g * 16) * _SEQ

        def lbody(l, accs):
            a0, a1, a2 = accs
            idx = plsc.load_gather(xv, [pos_vec + l])
            a0 = a0 + plsc.load_gather(t0v, [idx])
            a1 = a1 + plsc.load_gather(t1v, [idx])
            a2 = a2 + plsc.load_gather(t2v, [idx])
            return (a0, a1, a2)

        z = jnp.zeros((16,), jnp.float32)
        a0, a1, a2 = lax.fori_loop(0, _SEQ, lbody, (z, z, z))
        o0v[pl.ds(g * 16, 16)] = a0
        o1v[pl.ds(g * 16, 16)] = a1
        o2v[pl.ds(g * 16, 16)] = a2

    pltpu.sync_copy(o0v, o0_hbm.at[pl.ds(base, _ROWS)])
    pltpu.sync_copy(o1v, o1_hbm.at[pl.ds(base, _ROWS)])
    pltpu.sync_copy(o2v, o2_hbm.at[pl.ds(base, _ROWS)])


@functools.cache
def _sc_gather_reduce():
    col = jax.ShapeDtypeStruct((_BATCH,), jnp.float32)
    return pl.kernel(
        _sc_body,
        out_type=(col, col, col),
        mesh=plsc.VectorSubcoreMesh(core_axis_name="c", subcore_axis_name="s",
                                    num_cores=_NC, num_subcores=_NS),
        compiler_params=pltpu.CompilerParams(needs_layout_passes=False),
        scratch_types=(
            pltpu.VMEM((_ROWS * _SEQ,), jnp.int32),
            pltpu.VMEM((_VOCAB,), jnp.float32),
            pltpu.VMEM((_VOCAB,), jnp.float32),
            pltpu.VMEM((_VOCAB,), jnp.float32),
            pltpu.VMEM((_ROWS,), jnp.float32),
            pltpu.VMEM((_ROWS,), jnp.float32),
            pltpu.VMEM((_ROWS,), jnp.float32),
        ),
    )


def kernel(x, emb, W, b):
    t = _make_table(emb, W, b)
    o0, o1, o2 = _sc_gather_reduce()(
        x.astype(jnp.int32).reshape(-1), t[:, 0], t[:, 1], t[:, 2])
    return jnp.stack([o0, o1, o2], axis=1)
